# baseline (speedup 1.0000x reference)
import jax
import jax.numpy as jnp
from jax import lax
from jax.experimental import pallas as pl
from jax.experimental.pallas import tpu as pltpu
from jax.experimental.pallas import tpu_sc as plsc

_VOCAB = 92553
_HIDDEN = 2048
_MAX_SEQ = 4096
_NC = 1
_NS = 16
_NW = _NC * _NS
_C = 16
_Q = _C // 2
_NCHUNK = _MAX_SEQ // _C
_CPW = _NCHUNK // _NW


def _emb_body(ids_hbm, valid_hbm, embed_hbm, scale_hbm, zp_hbm, out_hbm,
              idx_v, rows_v, scl_v, zpv_v, val_v, out_v, zero_z,
              sem_in, sem_out, sem_z):
    wid = lax.axis_index("s") * _NC + lax.axis_index("c")
    out32_hbm = out_hbm.bitcast(jnp.int32)

    lanes = lax.iota(jnp.int32, 16)
    zvec = jnp.zeros((16,), jnp.int32)

    @plsc.parallel_loop(0, _Q * _HIDDEN // 16, 1, unroll=4)
    def zbody(i):
        zero_z[lax.div(i, _HIDDEN // 16), pl.ds(lax.rem(i, _HIDDEN // 16) * 16, 16)] = zvec

    def to_f16_bits(y):
        bits = plsc.bitcast(y, jnp.int32)
        sgn16 = lax.shift_right_logical(bits, 16) & 0x8000
        absb = bits & 0x7FFFFFFF
        h = lax.shift_right_logical(absb + 0x1000, 13)
        return h | sgn16

    chunks = [wid + k * _NW for k in range(_CPW)]
    anyv = []

    def load_valid(k):
        pltpu.sync_copy(valid_hbm.at[pl.ds(chunks[k] * _C, _C)], val_v.at[k & 1])
        anyv.append(val_v[k & 1][0] > 0.0)

    def issue_gathers(k):
        p = k & 1

        @pl.when(anyv[k])
        def _():
            pltpu.sync_copy(ids_hbm.at[pl.ds(chunks[k] * _C, _C)], idx_v.at[p])
            pltpu.async_copy(embed_hbm.at[idx_v.at[p]], rows_v.at[p], sem_in.at[p])
            pltpu.async_copy(scale_hbm.at[idx_v.at[p]], scl_v.at[p], sem_in.at[p])
            pltpu.async_copy(zp_hbm.at[idx_v.at[p]], zpv_v.at[p], sem_in.at[p])

    load_valid(0)
    issue_gathers(0)

    for k in range(_CPW):
        p = k & 1
        c = chunks[k]
        if k + 1 < _CPW:
            load_valid(k + 1)
            issue_gathers(k + 1)

        @pl.when(anyv[k])
        def _(k=k, p=p, c=c):
            pltpu.make_async_copy(embed_hbm.at[idx_v.at[p]], rows_v.at[p], sem_in.at[p]).wait()
            pltpu.make_async_copy(scale_hbm.at[idx_v.at[p]], scl_v.at[p], sem_in.at[p]).wait()
            pltpu.make_async_copy(zp_hbm.at[idx_v.at[p]], zpv_v.at[p], sem_in.at[p]).wait()
            if k >= 2:
                pltpu.make_async_copy(
                    out_v.at[p],
                    out32_hbm.at[pl.ds(chunks[k - 2] * _Q, _Q)],
                    sem_out.at[p]).wait()

            rebias = jnp.float32(2.0 ** -112)

            def row_body(q, _):
                r0full = jnp.full((16,), 2 * q, jnp.int32)
                r1full = r0full + 1
                vr0 = plsc.load_gather(val_v.at[p], [r0full]) * rebias
                s0 = plsc.load_gather(scl_v.at[p], [r0full]) * vr0
                z0 = plsc.load_gather(zpv_v.at[p], [r0full]) * vr0
                vr1 = plsc.load_gather(val_v.at[p], [r1full]) * rebias
                s1 = plsc.load_gather(scl_v.at[p], [r1full]) * vr1
                z1 = plsc.load_gather(zpv_v.at[p], [r1full]) * vr1

                @plsc.parallel_loop(0, _HIDDEN // 16, 1, unroll=8)
                def sbody(j):
                    col = j * 16
                    cidx = col + lanes
                    a = plsc.load_gather(rows_v.at[p], [r0full, cidx])
                    b = plsc.load_gather(rows_v.at[p], [r1full, cidx])
                    ha = to_f16_bits(a * s0 + z0)
                    hb = to_f16_bits(b * s1 + z1)
                    out_v[p, q, pl.ds(col, 16)] = ha | lax.shift_left(hb, 16)
                return 0
            lax.fori_loop(0, _Q, row_body, 0)

            pltpu.async_copy(out_v.at[p], out32_hbm.at[pl.ds(c * _Q, _Q)],
                             sem_out.at[p])

        @pl.when(jnp.logical_not(anyv[k]))
        def _(c=c):
            pltpu.async_copy(zero_z, out32_hbm.at[pl.ds(c * _Q, _Q)], sem_z)
            pltpu.make_async_copy(zero_z, out32_hbm.at[pl.ds(c * _Q, _Q)], sem_z).wait()

    for k in range(_CPW):
        p = k & 1
        later = anyv[k + 2] if k + 2 < _CPW else None
        cond = anyv[k] if later is None else jnp.logical_and(anyv[k], jnp.logical_not(later))

        @pl.when(cond)
        def _(k=k, p=p):
            pltpu.make_async_copy(
                out_v.at[p],
                out32_hbm.at[pl.ds(chunks[k] * _Q, _Q)],
                sem_out.at[p]).wait()


@jax.jit
def _emb_call(ids, valid, embed_data, scale, zp):
    fn = pl.kernel(
        _emb_body,
        out_type=jax.ShapeDtypeStruct((_MAX_SEQ, _HIDDEN), jnp.float16),
        mesh=plsc.VectorSubcoreMesh(core_axis_name="c", subcore_axis_name="s", num_cores=1),
        compiler_params=pltpu.CompilerParams(needs_layout_passes=False),
        scratch_types=[
            pltpu.VMEM((2, _C), jnp.int32),
            pltpu.VMEM((2, _C, _HIDDEN), jnp.float32),
            pltpu.VMEM((2, _C), jnp.float32),
            pltpu.VMEM((2, _C), jnp.float32),
            pltpu.VMEM((2, _C), jnp.float32),
            pltpu.VMEM((2, _Q, _HIDDEN), jnp.int32),
            pltpu.VMEM((_Q, _HIDDEN), jnp.int32),
            pltpu.SemaphoreType.DMA((2,)),
            pltpu.SemaphoreType.DMA((2,)),
            pltpu.SemaphoreType.DMA,
        ],
    )
    return fn(ids, valid, embed_data, scale, zp)


def kernel(input_ids, ids_len, embed_data, scale, zero_point):
    ids = input_ids.reshape(_MAX_SEQ)
    valid = (jnp.arange(_MAX_SEQ, dtype=jnp.int32) < ids_len).astype(jnp.float32)
    out = _emb_call(ids, valid, embed_data,
                    scale.reshape(_VOCAB), zero_point.reshape(_VOCAB))
    return out.reshape(1, _MAX_SEQ, _HIDDEN)


# speedup vs baseline: 1.4870x; 1.4870x over previous
import jax
import jax.numpy as jnp
from jax import lax
from jax.experimental import pallas as pl
from jax.experimental.pallas import tpu as pltpu
from jax.experimental.pallas import tpu_sc as plsc

_VOCAB = 92553
_HIDDEN = 2048
_MAX_SEQ = 4096
_NC = 2
_NS = 16
_NW = _NC * _NS
_C = 16
_Q = _C // 2
_NCHUNK = _MAX_SEQ // _C
_CPW = _NCHUNK // _NW


def _emb_body(ids_hbm, valid_hbm, embed_hbm, scale_hbm, zp_hbm, out_hbm,
              idx_v, rows_v, scl_v, zpv_v, val_v, out_v, zero_z,
              sem_in, sem_out, sem_z):
    wid = lax.axis_index("s") * _NC + lax.axis_index("c")
    out32_hbm = out_hbm.bitcast(jnp.int32)

    lanes = lax.iota(jnp.int32, 16)
    zvec = jnp.zeros((16,), jnp.int32)

    @plsc.parallel_loop(0, _Q * _HIDDEN // 16, 1, unroll=4)
    def zbody(i):
        zero_z[lax.div(i, _HIDDEN // 16), pl.ds(lax.rem(i, _HIDDEN // 16) * 16, 16)] = zvec

    def to_f16_bits(y):
        bits = plsc.bitcast(y, jnp.int32)
        sgn16 = lax.shift_right_logical(bits, 16) & 0x8000
        absb = bits & 0x7FFFFFFF
        h = lax.shift_right_logical(absb + 0x1000, 13)
        return h | sgn16

    chunks = [wid + k * _NW for k in range(_CPW)]
    anyv = []

    def load_valid(k):
        pltpu.sync_copy(valid_hbm.at[pl.ds(chunks[k] * _C, _C)], val_v.at[k & 1])
        anyv.append(val_v[k & 1][0] > 0.0)

    def issue_gathers(k):
        p = k & 1

        @pl.when(anyv[k])
        def _():
            pltpu.sync_copy(ids_hbm.at[pl.ds(chunks[k] * _C, _C)], idx_v.at[p])
            pltpu.async_copy(embed_hbm.at[idx_v.at[p]], rows_v.at[p], sem_in.at[p])
            pltpu.async_copy(scale_hbm.at[idx_v.at[p]], scl_v.at[p], sem_in.at[p])
            pltpu.async_copy(zp_hbm.at[idx_v.at[p]], zpv_v.at[p], sem_in.at[p])

    load_valid(0)
    issue_gathers(0)

    for k in range(_CPW):
        p = k & 1
        c = chunks[k]
        if k + 1 < _CPW:
            load_valid(k + 1)
            issue_gathers(k + 1)

        @pl.when(anyv[k])
        def _(k=k, p=p, c=c):
            pltpu.make_async_copy(embed_hbm.at[idx_v.at[p]], rows_v.at[p], sem_in.at[p]).wait()
            pltpu.make_async_copy(scale_hbm.at[idx_v.at[p]], scl_v.at[p], sem_in.at[p]).wait()
            pltpu.make_async_copy(zp_hbm.at[idx_v.at[p]], zpv_v.at[p], sem_in.at[p]).wait()
            if k >= 2:
                pltpu.make_async_copy(
                    out_v.at[p],
                    out32_hbm.at[pl.ds(chunks[k - 2] * _Q, _Q)],
                    sem_out.at[p]).wait()

            rebias = jnp.float32(2.0 ** -112)

            def row_body(q, _):
                r0full = jnp.full((16,), 2 * q, jnp.int32)
                r1full = r0full + 1
                vr0 = plsc.load_gather(val_v.at[p], [r0full]) * rebias
                s0 = plsc.load_gather(scl_v.at[p], [r0full]) * vr0
                z0 = plsc.load_gather(zpv_v.at[p], [r0full]) * vr0
                vr1 = plsc.load_gather(val_v.at[p], [r1full]) * rebias
                s1 = plsc.load_gather(scl_v.at[p], [r1full]) * vr1
                z1 = plsc.load_gather(zpv_v.at[p], [r1full]) * vr1

                @plsc.parallel_loop(0, _HIDDEN // 16, 1, unroll=8)
                def sbody(j):
                    col = j * 16
                    cidx = col + lanes
                    a = plsc.load_gather(rows_v.at[p], [r0full, cidx])
                    b = plsc.load_gather(rows_v.at[p], [r1full, cidx])
                    ha = to_f16_bits(a * s0 + z0)
                    hb = to_f16_bits(b * s1 + z1)
                    out_v[p, q, pl.ds(col, 16)] = ha | lax.shift_left(hb, 16)
                return 0
            lax.fori_loop(0, _Q, row_body, 0)

            pltpu.async_copy(out_v.at[p], out32_hbm.at[pl.ds(c * _Q, _Q)],
                             sem_out.at[p])

        @pl.when(jnp.logical_not(anyv[k]))
        def _(c=c):
            pltpu.async_copy(zero_z, out32_hbm.at[pl.ds(c * _Q, _Q)], sem_z)

    for k in range(_CPW):
        @pl.when(jnp.logical_not(anyv[k]))
        def _(k=k):
            pltpu.make_async_copy(
                zero_z, out32_hbm.at[pl.ds(chunks[k] * _Q, _Q)], sem_z).wait()

    for k in range(_CPW):
        p = k & 1
        later = anyv[k + 2] if k + 2 < _CPW else None
        cond = anyv[k] if later is None else jnp.logical_and(anyv[k], jnp.logical_not(later))

        @pl.when(cond)
        def _(k=k, p=p):
            pltpu.make_async_copy(
                out_v.at[p],
                out32_hbm.at[pl.ds(chunks[k] * _Q, _Q)],
                sem_out.at[p]).wait()


@jax.jit
def _emb_call(ids, valid, embed_data, scale, zp):
    fn = pl.kernel(
        _emb_body,
        out_type=jax.ShapeDtypeStruct((_MAX_SEQ, _HIDDEN), jnp.float16),
        mesh=plsc.VectorSubcoreMesh(core_axis_name="c", subcore_axis_name="s"),
        compiler_params=pltpu.CompilerParams(needs_layout_passes=False),
        scratch_types=[
            pltpu.VMEM((2, _C), jnp.int32),
            pltpu.VMEM((2, _C, _HIDDEN), jnp.float32),
            pltpu.VMEM((2, _C), jnp.float32),
            pltpu.VMEM((2, _C), jnp.float32),
            pltpu.VMEM((2, _C), jnp.float32),
            pltpu.VMEM((2, _Q, _HIDDEN), jnp.int32),
            pltpu.VMEM((_Q, _HIDDEN), jnp.int32),
            pltpu.SemaphoreType.DMA((2,)),
            pltpu.SemaphoreType.DMA((2,)),
            pltpu.SemaphoreType.DMA,
        ],
    )
    return fn(ids, valid, embed_data, scale, zp)


def kernel(input_ids, ids_len, embed_data, scale, zero_point):
    ids = input_ids.reshape(_MAX_SEQ)
    valid = (jnp.arange(_MAX_SEQ, dtype=jnp.int32) < ids_len).astype(jnp.float32)
    out = _emb_call(ids, valid, embed_data,
                    scale.reshape(_VOCAB), zero_point.reshape(_VOCAB))
    return out.reshape(1, _MAX_SEQ, _HIDDEN)
